# baseline (speedup 1.0000x reference)
import functools

import jax
import jax.numpy as jnp
from jax import lax
from jax.experimental import pallas as pl
from jax.experimental.pallas import tpu as pltpu
from jax.experimental.pallas import tpu_sc as plsc

_B = 512
_ROW = 4 * 64 * 64
_TABLE = 1000

_NC = 2
_NS = 16
_L = 16
_NW = _NC * _NS
_BPW = _B // _NW


def _sc_gather_body(t_hbm, sac_hbm, somac_hbm, a_hbm, b_hbm,
                    idx_v, sac_v, somac_v, a_v, b_v):
    wid = lax.axis_index("s") * _NC + lax.axis_index("c")
    base = wid * _BPW
    pltpu.sync_copy(sac_hbm, sac_v)
    pltpu.sync_copy(somac_hbm, somac_v)
    pltpu.sync_copy(t_hbm.at[pl.ds(base, _BPW)], idx_v)
    idx = idx_v[...]
    a_v[...] = plsc.load_gather(sac_v, [idx])
    b_v[...] = plsc.load_gather(somac_v, [idx])
    pltpu.sync_copy(a_v, a_hbm.at[pl.ds(base, _BPW)])
    pltpu.sync_copy(b_v, b_hbm.at[pl.ds(base, _BPW)])


@jax.jit
def _sc_gather(t, sac, somac):
    f = pl.kernel(
        _sc_gather_body,
        out_type=(
            jax.ShapeDtypeStruct((_B,), jnp.float32),
            jax.ShapeDtypeStruct((_B,), jnp.float32),
        ),
        mesh=plsc.VectorSubcoreMesh(core_axis_name="c", subcore_axis_name="s"),
        compiler_params=pltpu.CompilerParams(needs_layout_passes=False),
        scratch_types=[
            pltpu.VMEM((_BPW,), jnp.int32),
            pltpu.VMEM((_TABLE,), jnp.float32),
            pltpu.VMEM((_TABLE,), jnp.float32),
            pltpu.VMEM((_BPW,), jnp.float32),
            pltpu.VMEM((_BPW,), jnp.float32),
        ],
    )
    return f(t, sac, somac)


def _tc_body(a_ref, b_ref, x_ref, n_ref, o_ref):
    o_ref[...] = a_ref[...] * x_ref[...] + b_ref[...] * n_ref[...]


_ROWS_PER_BLOCK = 64


@functools.partial(jax.jit, static_argnames=())
def _tc_scale_add(a, b, x2d, n2d):
    grid = (_B // _ROWS_PER_BLOCK,)
    return pl.pallas_call(
        _tc_body,
        grid=grid,
        in_specs=[
            pl.BlockSpec((_ROWS_PER_BLOCK, 1), lambda i: (i, 0)),
            pl.BlockSpec((_ROWS_PER_BLOCK, 1), lambda i: (i, 0)),
            pl.BlockSpec((_ROWS_PER_BLOCK, _ROW), lambda i: (i, 0)),
            pl.BlockSpec((_ROWS_PER_BLOCK, _ROW), lambda i: (i, 0)),
        ],
        out_specs=pl.BlockSpec((_ROWS_PER_BLOCK, _ROW), lambda i: (i, 0)),
        out_shape=jax.ShapeDtypeStruct((_B, _ROW), jnp.float32),
        compiler_params=pltpu.CompilerParams(
            dimension_semantics=("arbitrary",),
        ),
    )(a, b, x2d, n2d)


def kernel(x_0, noise, t, sqrt_alphas_cumprod, sqrt_one_minus_alphas_cumprod):
    shape = x_0.shape
    t32 = t.astype(jnp.int32)
    a, b = _sc_gather(t32, sqrt_alphas_cumprod, sqrt_one_minus_alphas_cumprod)
    x2d = x_0.reshape(_B, _ROW)
    n2d = noise.reshape(_B, _ROW)
    out = _tc_scale_add(a.reshape(_B, 1), b.reshape(_B, 1), x2d, n2d)
    return (out.reshape(shape), noise)
